# baseline (speedup 1.0000x reference)
import math

import jax
import jax.numpy as jnp
from jax import lax
from jax.experimental import pallas as pl
from jax.experimental.pallas import tpu as pltpu


def _gate_body(S, E, CAP, B, NB):
    def body(x_ref, wt_ref, comb_ref, disp_ref, laux_ref, cnt_ref, me_ref, ce_ref):
        i = pl.program_id(0)

        @pl.when(i == 0)
        def _init():
            cnt_ref[...] = jnp.zeros_like(cnt_ref)
            me_ref[...] = jnp.zeros_like(me_ref)
            ce_ref[...] = jnp.zeros_like(ce_ref)

        x = x_ref[...]
        wt = wt_ref[...]
        logits = jnp.dot(x, wt, preferred_element_type=jnp.float32)

        m = jnp.max(logits, axis=1, keepdims=True)
        ex = jnp.exp(logits - m)
        den = jnp.sum(ex, axis=1, keepdims=True)
        gates = ex / den

        gmax = jnp.max(gates, axis=1, keepdims=True)
        eiota = lax.broadcasted_iota(jnp.int32, (B, E), 1)
        is_max = gates == gmax
        eidx = jnp.min(jnp.where(is_max, eiota, E), axis=1, keepdims=True)
        emask = eiota == eidx
        onehot = emask.astype(jnp.float32)

        r = lax.broadcasted_iota(jnp.int32, (B, B), 0)
        c = lax.broadcasted_iota(jnp.int32, (B, B), 1)
        tri = (c <= r).astype(jnp.float32)
        loc_incl = jnp.dot(tri, onehot, preferred_element_type=jnp.float32)

        prev = cnt_ref[...]
        loc = loc_incl - 1.0 + prev
        cnt_ref[...] = prev + loc_incl[B - 1:B, :]

        me_ref[...] += jnp.sum(gates, axis=0, keepdims=True)
        ce_ref[...] += loc_incl[B - 1:B, :]

        locs_tok = jnp.sum(loc * onehot, axis=1, keepdims=True)
        keep = locs_tok < float(CAP)

        tgt = eidx * CAP + locs_tok.astype(jnp.int32)
        jiota = lax.broadcasted_iota(jnp.int32, (B, E * CAP), 1)
        hit = (jiota == tgt) & keep
        comb_ref[...] = jnp.where(hit, gmax, 0.0)
        disp_ref[...] = hit

        @pl.when(i == NB - 1)
        def _fin():
            me = me_ref[...] / float(S)
            ce = ce_ref[...] / float(S)
            laux_ref[...] = jnp.sum(me * ce).reshape(1, 1) * float(E)

    return body


def kernel(input, W):
    S, D = input.shape
    E = W.shape[0]
    CAP = int(math.ceil(S / E))
    B = 128
    NB = S // B

    wt = W.T

    comb, disp, laux = pl.pallas_call(
        _gate_body(S, E, CAP, B, NB),
        grid=(NB,),
        in_specs=[
            pl.BlockSpec((B, D), lambda i: (i, 0)),
            pl.BlockSpec((D, E), lambda i: (0, 0)),
        ],
        out_specs=[
            pl.BlockSpec((B, E * CAP), lambda i: (i, 0)),
            pl.BlockSpec((B, E * CAP), lambda i: (i, 0)),
            pl.BlockSpec((1, 1), lambda i: (0, 0)),
        ],
        out_shape=[
            jax.ShapeDtypeStruct((S, E * CAP), jnp.float32),
            jax.ShapeDtypeStruct((S, E * CAP), jnp.bool_),
            jax.ShapeDtypeStruct((1, 1), jnp.float32),
        ],
        scratch_shapes=[
            pltpu.VMEM((1, E), jnp.float32),
            pltpu.VMEM((1, E), jnp.float32),
            pltpu.VMEM((1, E), jnp.float32),
        ],
    )(input, wt)

    return (laux.reshape(()), comb.reshape(S, E, CAP), disp.reshape(S, E, CAP))


# speedup vs baseline: 1.9926x; 1.9926x over previous
import math

import jax
import jax.numpy as jnp
from jax import lax
from jax.experimental import pallas as pl
from jax.experimental.pallas import tpu as pltpu


def _gate_body(S, E, CAP, B, NB):
    def body(x_ref, wt_ref, comb_ref, disp_ref, laux_ref, cnt_ref, me_ref, ce_ref):
        i = pl.program_id(0)

        @pl.when(i == 0)
        def _init():
            cnt_ref[...] = jnp.zeros_like(cnt_ref)
            me_ref[...] = jnp.zeros_like(me_ref)
            ce_ref[...] = jnp.zeros_like(ce_ref)

        x = x_ref[...]
        wt = wt_ref[...]
        logits = jnp.dot(x, wt, preferred_element_type=jnp.float32)

        m = jnp.max(logits, axis=1, keepdims=True)
        ex = jnp.exp(logits - m)
        den = jnp.sum(ex, axis=1, keepdims=True)
        gates = ex / den

        gmax = jnp.max(gates, axis=1, keepdims=True)
        eiota = lax.broadcasted_iota(jnp.int32, (B, E), 1)
        is_max = gates == gmax
        eidx = jnp.min(jnp.where(is_max, eiota, E), axis=1, keepdims=True)
        emask = eiota == eidx
        onehot = emask.astype(jnp.float32)

        r = lax.broadcasted_iota(jnp.int32, (B, B), 0)
        c = lax.broadcasted_iota(jnp.int32, (B, B), 1)
        tri = (c <= r).astype(jnp.float32)
        loc_incl = jnp.dot(tri, onehot, preferred_element_type=jnp.float32)

        prev = cnt_ref[...]
        loc = loc_incl - 1.0 + prev
        cnt_ref[...] = prev + loc_incl[B - 1:B, :]

        me_ref[...] += jnp.sum(gates, axis=0, keepdims=True)
        ce_ref[...] += loc_incl[B - 1:B, :]

        locs_tok = jnp.sum(loc * onehot, axis=1, keepdims=True)
        keep = locs_tok < float(CAP)

        e3 = lax.broadcasted_iota(jnp.int32, (B, E, CAP), 1)
        c3 = lax.broadcasted_iota(jnp.int32, (B, E, CAP), 2)
        eidx3 = eidx.reshape(B, 1, 1)
        loc3 = locs_tok.astype(jnp.int32).reshape(B, 1, 1)
        keep3 = keep.reshape(B, 1, 1)
        gmax3 = gmax.reshape(B, 1, 1)
        hit = (e3 == eidx3) & (c3 == loc3) & keep3
        comb_ref[...] = jnp.where(hit, gmax3, 0.0)
        disp_ref[...] = hit

        @pl.when(i == NB - 1)
        def _fin():
            me = me_ref[...] / float(S)
            ce = ce_ref[...] / float(S)
            laux_ref[...] = jnp.sum(me * ce).reshape(1, 1) * float(E)

    return body


def kernel(input, W):
    S, D = input.shape
    E = W.shape[0]
    CAP = int(math.ceil(S / E))
    B = 128
    NB = S // B

    wt = W.T

    comb, disp, laux = pl.pallas_call(
        _gate_body(S, E, CAP, B, NB),
        grid=(NB,),
        in_specs=[
            pl.BlockSpec((B, D), lambda i: (i, 0)),
            pl.BlockSpec((D, E), lambda i: (0, 0)),
        ],
        out_specs=[
            pl.BlockSpec((B, E, CAP), lambda i: (i, 0, 0)),
            pl.BlockSpec((B, E, CAP), lambda i: (i, 0, 0)),
            pl.BlockSpec((1, 1), lambda i: (0, 0)),
        ],
        out_shape=[
            jax.ShapeDtypeStruct((S, E, CAP), jnp.float32),
            jax.ShapeDtypeStruct((S, E, CAP), jnp.bool_),
            jax.ShapeDtypeStruct((1, 1), jnp.float32),
        ],
        scratch_shapes=[
            pltpu.VMEM((1, E), jnp.float32),
            pltpu.VMEM((1, E), jnp.float32),
            pltpu.VMEM((1, E), jnp.float32),
        ],
    )(input, wt)

    return (laux.reshape(()), comb, disp)


# speedup vs baseline: 2.0529x; 1.0303x over previous
import math

import jax
import jax.numpy as jnp
from jax import lax
from jax.experimental import pallas as pl
from jax.experimental.pallas import tpu as pltpu


def _gate_body(S, E, CAP, B, NB):
    def body(x_ref, wt_ref, comb_ref, disp_ref, laux_ref, cnt_ref, me_ref, ce_ref):
        i = pl.program_id(0)

        @pl.when(i == 0)
        def _init():
            cnt_ref[...] = jnp.zeros_like(cnt_ref)
            me_ref[...] = jnp.zeros_like(me_ref)
            ce_ref[...] = jnp.zeros_like(ce_ref)

        x = x_ref[...]
        wt = wt_ref[...]
        logits = jnp.dot(x, wt, preferred_element_type=jnp.float32)

        m = jnp.max(logits, axis=1, keepdims=True)
        ex = jnp.exp(logits - m)
        den = jnp.sum(ex, axis=1, keepdims=True)
        gates = ex / den

        gmax = jnp.max(gates, axis=1, keepdims=True)
        eiota = lax.broadcasted_iota(jnp.int32, (B, E), 1)
        is_max = gates == gmax
        eidx = jnp.min(jnp.where(is_max, eiota, E), axis=1, keepdims=True)
        emask = eiota == eidx
        onehot = emask.astype(jnp.float32)

        r = lax.broadcasted_iota(jnp.int32, (B, B), 0)
        c = lax.broadcasted_iota(jnp.int32, (B, B), 1)
        tri = (c <= r).astype(jnp.float32)
        loc_incl = jnp.dot(tri, onehot, preferred_element_type=jnp.float32)

        prev = cnt_ref[...]
        loc = loc_incl - 1.0 + prev
        cnt_ref[...] = prev + loc_incl[B - 1:B, :]

        me_ref[...] += jnp.sum(gates, axis=0, keepdims=True)
        ce_ref[...] += loc_incl[B - 1:B, :]

        locs_tok = jnp.sum(loc * onehot, axis=1, keepdims=True)
        e3 = lax.broadcasted_iota(jnp.int32, (B, E, CAP), 1)
        c3 = lax.broadcasted_iota(jnp.int32, (B, E, CAP), 2)
        eidx3 = eidx.reshape(B, 1, 1)
        loc3 = locs_tok.astype(jnp.int32).reshape(B, 1, 1)
        gmax3 = gmax.reshape(B, 1, 1)
        hit = (e3 == eidx3) & (c3 == loc3)
        comb_ref[...] = jnp.where(hit, gmax3, 0.0)
        disp_ref[...] = hit

        @pl.when(i == NB - 1)
        def _fin():
            me = me_ref[...] / float(S)
            ce = ce_ref[...] / float(S)
            laux_ref[...] = jnp.sum(me * ce).reshape(1, 1) * float(E)

    return body


def kernel(input, W):
    S, D = input.shape
    E = W.shape[0]
    CAP = int(math.ceil(S / E))
    B = 256
    NB = S // B

    wt = W.T

    comb, disp, laux = pl.pallas_call(
        _gate_body(S, E, CAP, B, NB),
        grid=(NB,),
        in_specs=[
            pl.BlockSpec((B, D), lambda i: (i, 0)),
            pl.BlockSpec((D, E), lambda i: (0, 0)),
        ],
        out_specs=[
            pl.BlockSpec((B, E, CAP), lambda i: (i, 0, 0)),
            pl.BlockSpec((B, E, CAP), lambda i: (i, 0, 0)),
            pl.BlockSpec((1, 1), lambda i: (0, 0)),
        ],
        out_shape=[
            jax.ShapeDtypeStruct((S, E, CAP), jnp.float32),
            jax.ShapeDtypeStruct((S, E, CAP), jnp.bool_),
            jax.ShapeDtypeStruct((1, 1), jnp.float32),
        ],
        scratch_shapes=[
            pltpu.VMEM((1, E), jnp.float32),
            pltpu.VMEM((1, E), jnp.float32),
            pltpu.VMEM((1, E), jnp.float32),
        ],
    )(input, wt)

    return (laux.reshape(()), comb, disp)


# speedup vs baseline: 3.1383x; 1.5287x over previous
import math

import jax
import jax.numpy as jnp
from jax import lax
from jax.experimental import pallas as pl
from jax.experimental.pallas import tpu as pltpu


def _gate_body(S, E, CAP, B, NB):
    def body(x_ref, wt_ref, comb_ref, disp_ref, laux_ref, cnt_ref, me_ref, ce_ref):
        i = pl.program_id(0)

        @pl.when(i == 0)
        def _init():
            cnt_ref[...] = jnp.zeros_like(cnt_ref)
            me_ref[...] = jnp.zeros_like(me_ref)
            ce_ref[...] = jnp.zeros_like(ce_ref)

        x = x_ref[...]
        wt = wt_ref[...]
        logits = jnp.dot(x, wt, preferred_element_type=jnp.float32)

        m = jnp.max(logits, axis=1, keepdims=True)
        ex = jnp.exp(logits - m)
        den = jnp.sum(ex, axis=1, keepdims=True)
        gates = ex / den

        gmax = jnp.max(gates, axis=1, keepdims=True)
        eiota = lax.broadcasted_iota(jnp.int32, (B, E), 1)
        is_max = gates == gmax
        eidx = jnp.min(jnp.where(is_max, eiota, E), axis=1, keepdims=True)
        emask = eiota == eidx
        onehot = emask.astype(jnp.float32)

        r = lax.broadcasted_iota(jnp.int32, (B, B), 0)
        c = lax.broadcasted_iota(jnp.int32, (B, B), 1)
        tri = (c <= r).astype(jnp.float32)
        loc_incl = jnp.dot(tri, onehot, preferred_element_type=jnp.float32)

        prev = cnt_ref[...]
        loc = loc_incl - 1.0 + prev
        cnt_ref[...] = prev + loc_incl[B - 1:B, :]

        me_ref[...] += jnp.sum(gates, axis=0, keepdims=True)
        ce_ref[...] += loc_incl[B - 1:B, :]

        locs_tok = jnp.sum(loc * onehot, axis=1, keepdims=True)
        e3 = lax.broadcasted_iota(jnp.int32, (B, E, CAP), 1)
        c3 = lax.broadcasted_iota(jnp.int32, (B, E, CAP), 2)
        eidx3 = eidx.reshape(B, 1, 1)
        loc3 = locs_tok.astype(jnp.int32).reshape(B, 1, 1)
        gmax3 = gmax.reshape(B, 1, 1)
        hit = (e3 == eidx3) & (c3 == loc3)
        comb_ref[...] = jnp.where(hit, gmax3, 0.0)
        disp_ref[...] = hit.astype(jnp.int8)

        @pl.when(i == NB - 1)
        def _fin():
            me = me_ref[...] / float(S)
            ce = ce_ref[...] / float(S)
            laux_ref[...] = jnp.sum(me * ce).reshape(1, 1) * float(E)

    return body


def kernel(input, W):
    S, D = input.shape
    E = W.shape[0]
    CAP = int(math.ceil(S / E))
    B = 256
    NB = S // B

    wt = W.T

    comb, disp, laux = pl.pallas_call(
        _gate_body(S, E, CAP, B, NB),
        grid=(NB,),
        in_specs=[
            pl.BlockSpec((B, D), lambda i: (i, 0)),
            pl.BlockSpec((D, E), lambda i: (0, 0)),
        ],
        out_specs=[
            pl.BlockSpec((B, E, CAP), lambda i: (i, 0, 0)),
            pl.BlockSpec((B, E, CAP), lambda i: (i, 0, 0)),
            pl.BlockSpec((1, 1), lambda i: (0, 0)),
        ],
        out_shape=[
            jax.ShapeDtypeStruct((S, E, CAP), jnp.float32),
            jax.ShapeDtypeStruct((S, E, CAP), jnp.int8),
            jax.ShapeDtypeStruct((1, 1), jnp.float32),
        ],
        scratch_shapes=[
            pltpu.VMEM((1, E), jnp.float32),
            pltpu.VMEM((1, E), jnp.float32),
            pltpu.VMEM((1, E), jnp.float32),
        ],
    )(input, wt)

    return (laux.reshape(()), comb, disp.view(jnp.bool_))
